# baseline (device time: 30285 ns/iter reference)
import jax
import jax.numpy as jnp
from jax import lax
from jax.experimental import pallas as pl
from jax.experimental.pallas import tpu as pltpu


def kernel(x, assign, W1, W2):
    T, D = x.shape
    E = W1.shape[0]
    assign2 = assign.reshape(T, 1)

    def body(x_ref, a_ref, w1_ref, w2_ref, out_ref,
             xs, xr, asnd, arcv, cs, cr, send_sems, recv_sems):
        my_x = lax.axis_index("x")
        my_y = lax.axis_index("y")
        peer = (my_x, 1 - my_y)

        barrier = pltpu.get_barrier_semaphore()
        pl.semaphore_signal(barrier, inc=1, device_id=peer,
                            device_id_type=pl.DeviceIdType.MESH)
        pl.semaphore_wait(barrier, 1)

        xs[...] = x_ref[...].astype(jnp.bfloat16)
        asnd[...] = a_ref[...]
        rx = pltpu.make_async_remote_copy(
            src_ref=xs, dst_ref=xr,
            send_sem=send_sems.at[0], recv_sem=recv_sems.at[0],
            device_id=peer, device_id_type=pl.DeviceIdType.MESH)
        ra = pltpu.make_async_remote_copy(
            src_ref=asnd, dst_ref=arcv,
            send_sem=send_sems.at[1], recv_sem=recv_sems.at[1],
            device_id=peer, device_id_type=pl.DeviceIdType.MESH)
        rx.start()
        ra.start()

        def moe(tok_bf, assign_col):
            acc = jnp.zeros((T, D), jnp.float32)
            for e in range(E):
                eg = my_y * E + e
                h = jnp.maximum(
                    jnp.dot(tok_bf, w1_ref[e].astype(jnp.bfloat16),
                            preferred_element_type=jnp.float32),
                    0.0).astype(jnp.bfloat16)
                o = jnp.dot(h, w2_ref[e].astype(jnp.bfloat16),
                            preferred_element_type=jnp.float32)
                acc = acc + jnp.where(assign_col == eg, o, 0.0)
            return acc

        out_ref[...] = moe(xs[...], a_ref[...])

        rx.wait()
        ra.wait()
        cs[...] = moe(xr[...], arcv[...]).astype(jnp.bfloat16)
        rc = pltpu.make_async_remote_copy(
            src_ref=cs, dst_ref=cr,
            send_sem=send_sems.at[2], recv_sem=recv_sems.at[2],
            device_id=peer, device_id_type=pl.DeviceIdType.MESH)
        rc.start()
        rc.wait()
        out_ref[...] = out_ref[...] + cr[...].astype(jnp.float32)

    return pl.pallas_call(
        body,
        out_shape=jax.ShapeDtypeStruct((T, D), jnp.float32),
        in_specs=[pl.BlockSpec(memory_space=pltpu.VMEM)] * 4,
        out_specs=pl.BlockSpec(memory_space=pltpu.VMEM),
        scratch_shapes=[
            pltpu.VMEM((T, D), jnp.bfloat16),
            pltpu.VMEM((T, D), jnp.bfloat16),
            pltpu.VMEM((T, 1), jnp.int32),
            pltpu.VMEM((T, 1), jnp.int32),
            pltpu.VMEM((T, D), jnp.bfloat16),
            pltpu.VMEM((T, D), jnp.bfloat16),
            pltpu.SemaphoreType.DMA((3,)),
            pltpu.SemaphoreType.DMA((3,)),
        ],
        compiler_params=pltpu.CompilerParams(collective_id=0),
    )(x, assign2, W1, W2)


# device time: 27792 ns/iter; 1.0897x vs baseline; 1.0897x over previous
import jax
import jax.numpy as jnp
from jax import lax
from jax.experimental import pallas as pl
from jax.experimental.pallas import tpu as pltpu


N_CHUNK = 2


def kernel(x, assign, W1, W2):
    T, D = x.shape
    E, _, F = W1.shape
    assign2 = assign.reshape(T, 1)
    Tc = T // N_CHUNK

    def body(x_ref, a_ref, w1_ref, w2_ref, out_ref,
             xs, xr, asnd, arcv, cs, cr, w1b, w2b, send_sems, recv_sems):
        my_x = lax.axis_index("x")
        my_y = lax.axis_index("y")
        peer = (my_x, 1 - my_y)

        barrier = pltpu.get_barrier_semaphore()
        pl.semaphore_signal(barrier, inc=1, device_id=peer,
                            device_id_type=pl.DeviceIdType.MESH)
        pl.semaphore_wait(barrier, 1)

        xs[...] = x_ref[...].astype(jnp.bfloat16)
        asnd[...] = a_ref[...]
        rx = pltpu.make_async_remote_copy(
            src_ref=xs, dst_ref=xr,
            send_sem=send_sems.at[0], recv_sem=recv_sems.at[0],
            device_id=peer, device_id_type=pl.DeviceIdType.MESH)
        ra = pltpu.make_async_remote_copy(
            src_ref=asnd, dst_ref=arcv,
            send_sem=send_sems.at[1], recv_sem=recv_sems.at[1],
            device_id=peer, device_id_type=pl.DeviceIdType.MESH)
        rx.start()
        ra.start()

        w1b[...] = w1_ref[...].astype(jnp.bfloat16)
        w2b[...] = w2_ref[...].astype(jnp.bfloat16)

        def moe(tok_bf, assign_col):
            n = tok_bf.shape[0]
            acc = jnp.zeros((n, D), jnp.float32)
            for e in range(E):
                eg = my_y * E + e
                h = jnp.maximum(
                    jnp.dot(tok_bf, w1b[e],
                            preferred_element_type=jnp.float32),
                    0.0).astype(jnp.bfloat16)
                o = jnp.dot(h, w2b[e], preferred_element_type=jnp.float32)
                acc = acc + jnp.where(assign_col == eg, o, 0.0)
            return acc

        out_ref[...] = moe(xs[...], a_ref[...])

        rx.wait()
        ra.wait()
        rcs = []
        for c in range(N_CHUNK):
            sl = pl.ds(c * Tc, Tc)
            cs[sl, :] = moe(xr[sl, :], arcv[sl, :]).astype(jnp.bfloat16)
            rc = pltpu.make_async_remote_copy(
                src_ref=cs.at[sl, :], dst_ref=cr.at[sl, :],
                send_sem=send_sems.at[2 + c], recv_sem=recv_sems.at[2 + c],
                device_id=peer, device_id_type=pl.DeviceIdType.MESH)
            rc.start()
            rcs.append(rc)
        for rc in rcs:
            rc.wait()
        out_ref[...] = out_ref[...] + cr[...].astype(jnp.float32)

    return pl.pallas_call(
        body,
        out_shape=jax.ShapeDtypeStruct((T, D), jnp.float32),
        in_specs=[pl.BlockSpec(memory_space=pltpu.VMEM)] * 4,
        out_specs=pl.BlockSpec(memory_space=pltpu.VMEM),
        scratch_shapes=[
            pltpu.VMEM((T, D), jnp.bfloat16),
            pltpu.VMEM((T, D), jnp.bfloat16),
            pltpu.VMEM((T, 1), jnp.int32),
            pltpu.VMEM((T, 1), jnp.int32),
            pltpu.VMEM((T, D), jnp.bfloat16),
            pltpu.VMEM((T, D), jnp.bfloat16),
            pltpu.VMEM((E, D, F), jnp.bfloat16),
            pltpu.VMEM((E, F, D), jnp.bfloat16),
            pltpu.SemaphoreType.DMA((2 + N_CHUNK,)),
            pltpu.SemaphoreType.DMA((2 + N_CHUNK,)),
        ],
        compiler_params=pltpu.CompilerParams(collective_id=0),
    )(x, assign2, W1, W2)


# device time: 23780 ns/iter; 1.2735x vs baseline; 1.1687x over previous
import jax
import jax.numpy as jnp
from jax import lax
from jax.experimental import pallas as pl
from jax.experimental.pallas import tpu as pltpu

SC = 2


def kernel(x, assign, W1, W2):
    T, D = x.shape
    E, _, F = W1.shape
    H = T // 2
    C = H // SC
    assign2 = assign.reshape(T, 1)

    def body(x_ref, a_ref, w1_ref, w2_ref, out_ref,
             xs, xr, asnd, arcv, cs, crd, crf, w1b, w2b,
             send_sems, recv_sems):
        my_x = lax.axis_index("x")
        my_y = lax.axis_index("y")
        ypeer = (my_x, 1 - my_y)
        xpeer = (1 - my_x, my_y)
        base = my_x * H
        obase = (1 - my_x) * H

        barrier = pltpu.get_barrier_semaphore()
        for nbr in (ypeer, xpeer):
            pl.semaphore_signal(barrier, inc=1, device_id=nbr,
                                device_id_type=pl.DeviceIdType.MESH)
        pl.semaphore_wait(barrier, 2)

        xs[...] = x_ref[...].astype(jnp.bfloat16)
        asnd[...] = a_ref[...]
        ra = pltpu.make_async_remote_copy(
            src_ref=asnd.at[pl.ds(base, H), :], dst_ref=arcv,
            send_sem=send_sems.at[0], recv_sem=recv_sems.at[0],
            device_id=ypeer, device_id_type=pl.DeviceIdType.MESH)
        ra.start()
        rxs = []
        for c in range(SC):
            rx = pltpu.make_async_remote_copy(
                src_ref=xs.at[pl.ds(base + c * C, C), :],
                dst_ref=xr.at[pl.ds(c * C, C), :],
                send_sem=send_sems.at[1 + c], recv_sem=recv_sems.at[1 + c],
                device_id=ypeer, device_id_type=pl.DeviceIdType.MESH)
            rx.start()
            rxs.append(rx)

        w1b[...] = w1_ref[...].astype(jnp.bfloat16)
        w2b[...] = w2_ref[...].astype(jnp.bfloat16)

        def moe(tok_bf, assign_col):
            n = tok_bf.shape[0]
            acc = jnp.zeros((n, D), jnp.float32)
            for e in range(E):
                eg = my_y * E + e
                h = jnp.maximum(
                    jnp.dot(tok_bf, w1b[e],
                            preferred_element_type=jnp.float32),
                    0.0).astype(jnp.bfloat16)
                o = jnp.dot(h, w2b[e], preferred_element_type=jnp.float32)
                acc = acc + jnp.where(assign_col == eg, o, 0.0)
            return acc

        out_ref[...] = moe(xs[...], a_ref[...])

        ra.wait()
        rcs = []
        for c in range(SC):
            sl = pl.ds(c * C, C)
            rxs[c].wait()
            cs[sl, :] = moe(xr[sl, :], arcv[sl, :]).astype(jnp.bfloat16)
            rc = pltpu.make_async_remote_copy(
                src_ref=cs.at[sl, :], dst_ref=crd.at[sl, :],
                send_sem=send_sems.at[1 + SC + c],
                recv_sem=recv_sems.at[1 + SC + c],
                device_id=ypeer, device_id_type=pl.DeviceIdType.MESH)
            rc.start()
            rcs.append(rc)

        rfs = []
        for c in range(SC):
            sl = pl.ds(c * C, C)
            rcs[c].wait()
            rf = pltpu.make_async_remote_copy(
                src_ref=crd.at[sl, :], dst_ref=crf.at[sl, :],
                send_sem=send_sems.at[1 + 2 * SC + c],
                recv_sem=recv_sems.at[1 + 2 * SC + c],
                device_id=xpeer, device_id_type=pl.DeviceIdType.MESH)
            rf.start()
            rfs.append(rf)
            osl = pl.ds(base + c * C, C)
            out_ref[osl, :] = out_ref[osl, :] + crd[sl, :].astype(jnp.float32)

        for c in range(SC):
            sl = pl.ds(c * C, C)
            rfs[c].wait()
            osl = pl.ds(obase + c * C, C)
            out_ref[osl, :] = out_ref[osl, :] + crf[sl, :].astype(jnp.float32)

    return pl.pallas_call(
        body,
        out_shape=jax.ShapeDtypeStruct((T, D), jnp.float32),
        in_specs=[pl.BlockSpec(memory_space=pltpu.VMEM)] * 4,
        out_specs=pl.BlockSpec(memory_space=pltpu.VMEM),
        scratch_shapes=[
            pltpu.VMEM((T, D), jnp.bfloat16),
            pltpu.VMEM((H, D), jnp.bfloat16),
            pltpu.VMEM((T, 1), jnp.int32),
            pltpu.VMEM((H, 1), jnp.int32),
            pltpu.VMEM((H, D), jnp.bfloat16),
            pltpu.VMEM((H, D), jnp.bfloat16),
            pltpu.VMEM((H, D), jnp.bfloat16),
            pltpu.VMEM((E, D, F), jnp.bfloat16),
            pltpu.VMEM((E, F, D), jnp.bfloat16),
            pltpu.SemaphoreType.DMA((1 + 3 * SC,)),
            pltpu.SemaphoreType.DMA((1 + 3 * SC,)),
        ],
        compiler_params=pltpu.CompilerParams(collective_id=0),
    )(x, assign2, W1, W2)
